# baseline (speedup 1.0000x reference)
import functools

import jax
import jax.numpy as jnp
from jax import lax
from jax.experimental import pallas as pl
from jax.experimental.pallas import tpu as pltpu
from jax.experimental.pallas import tpu_sc as plsc

B, N, S = 8, 4096, 1024
D1, D2 = 256, 512
HD = D2 // 2
BN_PTS = B * N
HB = B // 2
HPTS = HB * N
NT1 = 256
NT3 = 256
GW = 128
NB1 = N // NT1
NB3 = N // NT3
INV_CNT = 1.0 / float(BN_PTS)


def _k0_body(p2_ref, t_ref):
    t_ref[...] = jnp.transpose(p2_ref[0])


def _make_table(points2):
    return pl.pallas_call(
        _k0_body,
        grid=(B, 2),
        in_specs=[pl.BlockSpec((1, D2, S // 2), lambda b, t: (b, 0, t))],
        out_specs=pl.BlockSpec((S // 2, D2), lambda b, t: (2 * b + t, 0)),
        out_shape=jax.ShapeDtypeStruct((B * S, D2), jnp.float32),
    )(points2)


def _k1_body(x1_ref, x2_ref, idx_ref, w_ref, *, boff):
    b = pl.program_id(0)
    x1 = x1_ref[0]
    x2 = x2_ref[0]
    x1b = x1.astype(jnp.bfloat16).astype(jnp.float32)
    x2b = x2.astype(jnp.bfloat16).astype(jnp.float32)
    acc = x1b[:, 0:1] * x2b[0:1, :]
    acc = acc + x1b[:, 1:2] * x2b[1:2, :]
    acc = acc + x1b[:, 2:3] * x2b[2:3, :]
    x1sq = jnp.sum(x1 * x1, axis=1, keepdims=True)
    x2sq = jnp.sum(x2 * x2, axis=0, keepdims=True)
    d = (-2.0 * acc + x1sq) + x2sq
    iota = lax.broadcasted_iota(jnp.int32, (NT1, S), 1)
    big = jnp.float32(3.0e38)
    vals, ids = [], []
    work = d
    for _ in range(3):
        m = jnp.min(work, axis=1, keepdims=True)
        sel = jnp.where(work == m, iota, jnp.int32(S))
        ik = jnp.min(sel, axis=1, keepdims=True)
        vals.append(m)
        ids.append(ik)
        work = jnp.where(iota == ik, big, work)
    r = [1.0 / (v + 1e-8) for v in vals]
    rs = r[0] + r[1] + r[2]
    w0 = r[0] / rs
    w1 = r[1] / rs
    w2 = r[2] / rs
    wz = jnp.zeros_like(w0)
    w_ref[0] = jnp.concatenate([w0, w1, w2, wz], axis=1)
    base = (b + boff) * S
    cols = []
    for k in range(3):
        gidx = 2 * (ids[k] + base)
        cols.append(gidx)
        cols.append(gidx + 1)
    zc = jnp.zeros_like(cols[0])
    idx_ref[...] = jnp.transpose(
        jnp.concatenate(cols + [zc, zc], axis=1))


def _knn_weights(x1t, x2p, boff):
    return pl.pallas_call(
        functools.partial(_k1_body, boff=boff),
        grid=(HB, NB1),
        in_specs=[
            pl.BlockSpec((1, NT1, 3), lambda b, t: (b + boff, t, 0)),
            pl.BlockSpec((1, 8, S), lambda b, t: (b + boff, 0, 0)),
        ],
        out_specs=[
            pl.BlockSpec((8, NT1), lambda b, t: (0, b * NB1 + t)),
            pl.BlockSpec((1, NT1, 4), lambda b, t: (b, t, 0)),
        ],
        out_shape=[
            jax.ShapeDtypeStruct((8, HPTS), jnp.int32),
            jax.ShapeDtypeStruct((HB, N, 4), jnp.float32),
        ],
    )(x1t, x2p)


def _gather_rows(table2, idx8):
    nwin = HPTS // GW
    mesh = plsc.VectorSubcoreMesh(core_axis_name="c", subcore_axis_name="s")

    @pl.kernel(
        out_type=jax.ShapeDtypeStruct((6 * HPTS, HD), jnp.float32),
        mesh=mesh,
    )
    def k(tab_hbm, i_hbm, o_hbm):
        def body(i_vmem, o_vmem):
            pltpu.sync_copy(tab_hbm.at[i_vmem.at[0]], o_vmem)

        pltpu.emit_pipeline(
            body,
            grid=(6, nwin),
            in_specs=[pl.BlockSpec((1, GW), lambda q, i: (q, i))],
            out_specs=[pl.BlockSpec((GW, HD), lambda q, i: (q * nwin + i, 0))],
            core_axis_name=("c", "s"),
            dimension_semantics=(pltpu.PARALLEL, pltpu.PARALLEL),
        )(i_hbm, o_hbm)

    return k(table2, idx8)


def _k3_body(g0l_ref, g0h_ref, g1l_ref, g1h_ref, g2l_ref, g2h_ref,
             w_ref, p1_ref, w0a_ref, w0bp_ref, b0_ref, *rest):
    h0_ref, sums_ref = rest[-2], rest[-1]
    p1b = p1_ref[0].astype(jnp.bfloat16)
    h = jnp.dot(w0a_ref[...], p1b, preferred_element_type=jnp.float32)
    w = w_ref[0]
    ilo = w[:, 0:1] * g0l_ref[0]
    ilo = ilo + w[:, 1:2] * g1l_ref[0]
    ilo = ilo + w[:, 2:3] * g2l_ref[0]
    ihi = w[:, 0:1] * g0h_ref[0]
    ihi = ihi + w[:, 1:2] * g1h_ref[0]
    ihi = ihi + w[:, 2:3] * g2h_ref[0]
    interp = jnp.concatenate([ilo, ihi], axis=1).astype(jnp.bfloat16)
    h = h + lax.dot_general(
        w0bp_ref[...], interp,
        (((1,), (1,)), ((), ())), preferred_element_type=jnp.float32)
    h = h + b0_ref[...]
    h0_ref[...] = h.astype(jnp.bfloat16)

    @pl.when(jnp.logical_and(pl.program_id(0) == 0, pl.program_id(1) == 0))
    def _():
        sums_ref[...] = jnp.zeros_like(sums_ref)

    sums_ref[:, 0:1] += jnp.sum(h, axis=1, keepdims=True)
    sums_ref[:, 1:2] += jnp.sum(h * h, axis=1, keepdims=True)


def _mlp0(g6, w4, points1, w0a, w0bp, b0c, boff, h0_prev=None):
    gspec = [pl.BlockSpec((1, NT3, HD),
                          (lambda q: (lambda b, t: (q, b * NB3 + t, 0)))(q))
             for q in range(6)]
    in_specs = gspec + [
        pl.BlockSpec((1, NT3, 4), lambda b, t: (b, t, 0)),
        pl.BlockSpec((1, D1, NT3), lambda b, t: (b + boff, 0, t)),
        pl.BlockSpec((512, D1), lambda b, t: (0, 0)),
        pl.BlockSpec((512, D2), lambda b, t: (0, 0)),
        pl.BlockSpec((512, 1), lambda b, t: (0, 0)),
    ]
    args = [g6, g6, g6, g6, g6, g6, w4, points1, w0a, w0bp, b0c]
    aliases = {}
    if h0_prev is not None:
        in_specs.append(pl.BlockSpec(memory_space=pl.ANY))
        args.append(h0_prev)
        aliases = {len(args) - 1: 0}
    return pl.pallas_call(
        _k3_body,
        grid=(HB, NB3),
        in_specs=in_specs,
        out_specs=[
            pl.BlockSpec((512, NT3), lambda b, t: (0, (b + boff) * NB3 + t)),
            pl.BlockSpec((512, 8), lambda b, t: (0, 0)),
        ],
        out_shape=[
            jax.ShapeDtypeStruct((512, BN_PTS), jnp.bfloat16),
            jax.ShapeDtypeStruct((512, 8), jnp.float32),
        ],
        input_output_aliases=aliases,
    )(*args)


def _bn_ac(sa, sb, gam, bet):
    mean = (sa[:, 0:1] + sb[:, 0:1]) * INV_CNT
    ex2 = (sa[:, 1:2] + sb[:, 1:2]) * INV_CNT
    var = jnp.maximum(ex2 - mean * mean, 0.0)
    a = gam * lax.rsqrt(var + 1e-5)
    c = bet - mean * a
    return a, c


def _k4_body(h0_ref, sa_ref, sb_ref, gam_ref, bet_ref, w1_ref, b1_ref,
             h1_ref, sums_ref, ac_ref):
    @pl.when(pl.program_id(0) == 0)
    def _():
        a0, c0 = _bn_ac(sa_ref[...], sb_ref[...], gam_ref[...], bet_ref[...])
        ac_ref[:, 0:1] = a0
        ac_ref[:, 1:2] = c0

    a = ac_ref[:, 0:1]
    c = ac_ref[:, 1:2]
    h = h0_ref[...].astype(jnp.float32)
    x = jnp.maximum(a * h + c, 0.0)
    h1 = jnp.dot(w1_ref[...], x.astype(jnp.bfloat16),
                 preferred_element_type=jnp.float32)
    h1 = h1 + b1_ref[...]
    h1_ref[...] = h1.astype(jnp.bfloat16)

    @pl.when(pl.program_id(0) == 0)
    def _():
        sums_ref[...] = jnp.zeros_like(sums_ref)

    sums_ref[:, 0:1] += jnp.sum(h1, axis=1, keepdims=True)
    sums_ref[:, 1:2] += jnp.sum(h1 * h1, axis=1, keepdims=True)


def _mlp1(h0, s0a, s0b, gam0, bet0, w1, b1c):
    stat_spec = pl.BlockSpec((512, 8), lambda i: (0, 0))
    col_spec = pl.BlockSpec((512, 1), lambda i: (0, 0))
    return pl.pallas_call(
        _k4_body,
        grid=(BN_PTS // NT3,),
        in_specs=[
            pl.BlockSpec((512, NT3), lambda i: (0, i)),
            stat_spec, stat_spec, col_spec, col_spec,
            pl.BlockSpec((512, 512), lambda i: (0, 0)),
            col_spec,
        ],
        out_specs=[
            pl.BlockSpec((512, NT3), lambda i: (0, i)),
            pl.BlockSpec((512, 8), lambda i: (0, 0)),
        ],
        out_shape=[
            jax.ShapeDtypeStruct((512, BN_PTS), jnp.bfloat16),
            jax.ShapeDtypeStruct((512, 8), jnp.float32),
        ],
        scratch_shapes=[pltpu.VMEM((512, 2), jnp.float32)],
    )(h0, s0a, s0b, gam0, bet0, w1, b1c)


def _k5_body(h1_ref, s1_ref, gam_ref, bet_ref, out_ref, ac_ref):
    first = jnp.logical_and(pl.program_id(0) == 0, pl.program_id(1) == 0)

    @pl.when(first)
    def _():
        z = jnp.zeros_like(s1_ref[...])
        a1, c1 = _bn_ac(s1_ref[...], z, gam_ref[...], bet_ref[...])
        ac_ref[:, 0:1] = a1
        ac_ref[:, 1:2] = c1

    h = h1_ref[...].astype(jnp.float32)
    out_ref[0] = jnp.maximum(ac_ref[:, 0:1] * h + ac_ref[:, 1:2], 0.0)


def _final(h1, s1, gam1, bet1):
    col_spec = pl.BlockSpec((512, 1), lambda b, t: (0, 0))
    return pl.pallas_call(
        _k5_body,
        grid=(B, NB3),
        in_specs=[
            pl.BlockSpec((512, NT3), lambda b, t: (0, b * NB3 + t)),
            pl.BlockSpec((512, 8), lambda b, t: (0, 0)),
            col_spec, col_spec,
        ],
        out_specs=pl.BlockSpec((1, 512, NT3), lambda b, t: (b, 0, t)),
        out_shape=jax.ShapeDtypeStruct((B, 512, N), jnp.float32),
        scratch_shapes=[pltpu.VMEM((512, 2), jnp.float32)],
    )(h1, s1, gam1, bet1)


def kernel(xyz1, xyz2, points1, points2, W0, b0, g0, beta0,
           W1, b1, g1, beta1):
    x1t = jnp.transpose(xyz1, (0, 2, 1))
    x2p = jnp.concatenate(
        [xyz2, jnp.zeros((B, 5, S), xyz2.dtype)], axis=1)

    table = _make_table(points2)
    table2 = table.reshape(B * S * 2, HD)

    idx8a, w4a = _knn_weights(x1t, x2p, 0)
    ga = _gather_rows(table2, idx8a)
    idx8b, w4b = _knn_weights(x1t, x2p, HB)
    gb = _gather_rows(table2, idx8b)

    w0a = W0[:, :D1].astype(jnp.bfloat16)
    w0bp = W0[:, D1:].astype(jnp.bfloat16)
    b0c = b0.reshape(512, 1)

    h0a, s0a = _mlp0(ga.reshape(6, HPTS, HD), w4a, points1,
                     w0a, w0bp, b0c, 0)
    h0, s0b = _mlp0(gb.reshape(6, HPTS, HD), w4b, points1,
                    w0a, w0bp, b0c, HB, h0_prev=h0a)

    h1, s1 = _mlp1(h0, s0a, s0b, g0.reshape(512, 1), beta0.reshape(512, 1),
                   W1.astype(jnp.bfloat16), b1.reshape(512, 1))

    return _final(h1, s1, g1.reshape(512, 1), beta1.reshape(512, 1))


# speedup vs baseline: 1.0607x; 1.0607x over previous
import functools

import jax
import jax.numpy as jnp
from jax import lax
from jax.experimental import pallas as pl
from jax.experimental.pallas import tpu as pltpu
from jax.experimental.pallas import tpu_sc as plsc

B, N, S = 8, 4096, 1024
D1, D2 = 256, 512
HD = D2 // 2
BN_PTS = B * N
HB = B // 2
HPTS = HB * N
NT1 = 512
NT3 = 256
GW = 128
NB1 = N // NT1
NB3 = N // NT3
INV_CNT = 1.0 / float(BN_PTS)


def _k0_body(p2_ref, t_ref):
    t_ref[...] = jnp.transpose(p2_ref[0])


def _make_table(points2):
    return pl.pallas_call(
        _k0_body,
        grid=(B, 2),
        in_specs=[pl.BlockSpec((1, D2, S // 2), lambda b, t: (b, 0, t))],
        out_specs=pl.BlockSpec((S // 2, D2), lambda b, t: (2 * b + t, 0)),
        out_shape=jax.ShapeDtypeStruct((B * S, D2), jnp.float32),
    )(points2)


def _k1_body(x1_ref, x2_ref, idx_ref, w_ref, *, boff):
    b = pl.program_id(0)
    x1 = x1_ref[0]
    x2 = x2_ref[0]
    x1b = x1.astype(jnp.bfloat16).astype(jnp.float32)
    x2b = x2.astype(jnp.bfloat16).astype(jnp.float32)
    acc = x1b[:, 0:1] * x2b[0:1, :]
    acc = acc + x1b[:, 1:2] * x2b[1:2, :]
    acc = acc + x1b[:, 2:3] * x2b[2:3, :]
    x1sq = jnp.sum(x1 * x1, axis=1, keepdims=True)
    x2sq = jnp.sum(x2 * x2, axis=0, keepdims=True)
    d = (-2.0 * acc + x1sq) + x2sq
    iota = lax.broadcasted_iota(jnp.int32, (NT1, S), 1)
    big = jnp.float32(3.0e38)
    vals, ids = [], []
    work = d
    for _ in range(3):
        m = jnp.min(work, axis=1, keepdims=True)
        sel = jnp.where(work == m, iota, jnp.int32(S))
        ik = jnp.min(sel, axis=1, keepdims=True)
        vals.append(m)
        ids.append(ik)
        work = jnp.where(iota == ik, big, work)
    r = [1.0 / (v + 1e-8) for v in vals]
    rs = r[0] + r[1] + r[2]
    w0 = r[0] / rs
    w1 = r[1] / rs
    w2 = r[2] / rs
    wz = jnp.zeros_like(w0)
    w_ref[0] = jnp.concatenate([w0, w1, w2, wz], axis=1)
    base = (b + boff) * S
    cols = []
    for k in range(3):
        gidx = 2 * (ids[k] + base)
        cols.append(gidx)
        cols.append(gidx + 1)
    zc = jnp.zeros_like(cols[0])
    idx_ref[...] = jnp.transpose(
        jnp.concatenate(cols + [zc, zc], axis=1))


def _knn_weights(x1t, x2p, boff):
    return pl.pallas_call(
        functools.partial(_k1_body, boff=boff),
        grid=(HB, NB1),
        in_specs=[
            pl.BlockSpec((1, NT1, 3), lambda b, t: (b + boff, t, 0)),
            pl.BlockSpec((1, 8, S), lambda b, t: (b + boff, 0, 0)),
        ],
        out_specs=[
            pl.BlockSpec((8, NT1), lambda b, t: (0, b * NB1 + t)),
            pl.BlockSpec((1, NT1, 4), lambda b, t: (b, t, 0)),
        ],
        out_shape=[
            jax.ShapeDtypeStruct((8, HPTS), jnp.int32),
            jax.ShapeDtypeStruct((HB, N, 4), jnp.float32),
        ],
    )(x1t, x2p)


def _gather_rows(table2, idx8):
    nwin = HPTS // GW
    mesh = plsc.VectorSubcoreMesh(core_axis_name="c", subcore_axis_name="s")

    @pl.kernel(
        out_type=jax.ShapeDtypeStruct((6 * HPTS, HD), jnp.float32),
        mesh=mesh,
    )
    def k(tab_hbm, i_hbm, o_hbm):
        def body(i_vmem, o_vmem):
            pltpu.sync_copy(tab_hbm.at[i_vmem.at[0]], o_vmem)

        pltpu.emit_pipeline(
            body,
            grid=(6, nwin),
            in_specs=[pl.BlockSpec((1, GW), lambda q, i: (q, i))],
            out_specs=[pl.BlockSpec((GW, HD), lambda q, i: (q * nwin + i, 0))],
            core_axis_name=("c", "s"),
            dimension_semantics=(pltpu.PARALLEL, pltpu.PARALLEL),
        )(i_hbm, o_hbm)

    return k(table2, idx8)


def _k3_body(g0l_ref, g0h_ref, g1l_ref, g1h_ref, g2l_ref, g2h_ref,
             w_ref, p1_ref, w0a_ref, w0bp_ref, b0_ref, *rest):
    h0_ref, sums_ref = rest[-2], rest[-1]
    p1b = p1_ref[0].astype(jnp.bfloat16)
    h = jnp.dot(w0a_ref[...], p1b, preferred_element_type=jnp.float32)
    w = w_ref[0]
    ilo = w[:, 0:1] * g0l_ref[0]
    ilo = ilo + w[:, 1:2] * g1l_ref[0]
    ilo = ilo + w[:, 2:3] * g2l_ref[0]
    ihi = w[:, 0:1] * g0h_ref[0]
    ihi = ihi + w[:, 1:2] * g1h_ref[0]
    ihi = ihi + w[:, 2:3] * g2h_ref[0]
    interp = jnp.concatenate([ilo, ihi], axis=1).astype(jnp.bfloat16)
    h = h + lax.dot_general(
        w0bp_ref[...], interp,
        (((1,), (1,)), ((), ())), preferred_element_type=jnp.float32)
    h = h + b0_ref[...]
    h0_ref[...] = h.astype(jnp.bfloat16)

    @pl.when(jnp.logical_and(pl.program_id(0) == 0, pl.program_id(1) == 0))
    def _():
        sums_ref[...] = jnp.zeros_like(sums_ref)

    sums_ref[:, 0:1] += jnp.sum(h, axis=1, keepdims=True)
    sums_ref[:, 1:2] += jnp.sum(h * h, axis=1, keepdims=True)


def _mlp0(g6, w4, points1, w0a, w0bp, b0c, boff, h0_prev=None):
    gspec = [pl.BlockSpec((1, NT3, HD),
                          (lambda q: (lambda b, t: (q, b * NB3 + t, 0)))(q))
             for q in range(6)]
    in_specs = gspec + [
        pl.BlockSpec((1, NT3, 4), lambda b, t: (b, t, 0)),
        pl.BlockSpec((1, D1, NT3), lambda b, t: (b + boff, 0, t)),
        pl.BlockSpec((512, D1), lambda b, t: (0, 0)),
        pl.BlockSpec((512, D2), lambda b, t: (0, 0)),
        pl.BlockSpec((512, 1), lambda b, t: (0, 0)),
    ]
    args = [g6, g6, g6, g6, g6, g6, w4, points1, w0a, w0bp, b0c]
    aliases = {}
    if h0_prev is not None:
        in_specs.append(pl.BlockSpec(memory_space=pl.ANY))
        args.append(h0_prev)
        aliases = {len(args) - 1: 0}
    return pl.pallas_call(
        _k3_body,
        grid=(HB, NB3),
        in_specs=in_specs,
        out_specs=[
            pl.BlockSpec((512, NT3), lambda b, t: (0, (b + boff) * NB3 + t)),
            pl.BlockSpec((512, 8), lambda b, t: (0, 0)),
        ],
        out_shape=[
            jax.ShapeDtypeStruct((512, BN_PTS), jnp.bfloat16),
            jax.ShapeDtypeStruct((512, 8), jnp.float32),
        ],
        input_output_aliases=aliases,
    )(*args)


def _bn_ac(sa, sb, gam, bet):
    mean = (sa[:, 0:1] + sb[:, 0:1]) * INV_CNT
    ex2 = (sa[:, 1:2] + sb[:, 1:2]) * INV_CNT
    var = jnp.maximum(ex2 - mean * mean, 0.0)
    a = gam * lax.rsqrt(var + 1e-5)
    c = bet - mean * a
    return a, c


def _k4_body(h0_ref, sa_ref, sb_ref, gam_ref, bet_ref, w1_ref, b1_ref,
             h1_ref, sums_ref, ac_ref):
    @pl.when(pl.program_id(0) == 0)
    def _():
        a0, c0 = _bn_ac(sa_ref[...], sb_ref[...], gam_ref[...], bet_ref[...])
        ac_ref[:, 0:1] = a0
        ac_ref[:, 1:2] = c0

    a = ac_ref[:, 0:1]
    c = ac_ref[:, 1:2]
    h = h0_ref[...].astype(jnp.float32)
    x = jnp.maximum(a * h + c, 0.0)
    h1 = jnp.dot(w1_ref[...], x.astype(jnp.bfloat16),
                 preferred_element_type=jnp.float32)
    h1 = h1 + b1_ref[...]
    h1_ref[...] = h1.astype(jnp.bfloat16)

    @pl.when(pl.program_id(0) == 0)
    def _():
        sums_ref[...] = jnp.zeros_like(sums_ref)

    sums_ref[:, 0:1] += jnp.sum(h1, axis=1, keepdims=True)
    sums_ref[:, 1:2] += jnp.sum(h1 * h1, axis=1, keepdims=True)


def _mlp1(h0, s0a, s0b, gam0, bet0, w1, b1c):
    stat_spec = pl.BlockSpec((512, 8), lambda i: (0, 0))
    col_spec = pl.BlockSpec((512, 1), lambda i: (0, 0))
    return pl.pallas_call(
        _k4_body,
        grid=(BN_PTS // NT3,),
        in_specs=[
            pl.BlockSpec((512, NT3), lambda i: (0, i)),
            stat_spec, stat_spec, col_spec, col_spec,
            pl.BlockSpec((512, 512), lambda i: (0, 0)),
            col_spec,
        ],
        out_specs=[
            pl.BlockSpec((512, NT3), lambda i: (0, i)),
            pl.BlockSpec((512, 8), lambda i: (0, 0)),
        ],
        out_shape=[
            jax.ShapeDtypeStruct((512, BN_PTS), jnp.bfloat16),
            jax.ShapeDtypeStruct((512, 8), jnp.float32),
        ],
        scratch_shapes=[pltpu.VMEM((512, 2), jnp.float32)],
    )(h0, s0a, s0b, gam0, bet0, w1, b1c)


def _k5_body(h1_ref, s1_ref, gam_ref, bet_ref, out_ref, ac_ref):
    first = jnp.logical_and(pl.program_id(0) == 0, pl.program_id(1) == 0)

    @pl.when(first)
    def _():
        z = jnp.zeros_like(s1_ref[...])
        a1, c1 = _bn_ac(s1_ref[...], z, gam_ref[...], bet_ref[...])
        ac_ref[:, 0:1] = a1
        ac_ref[:, 1:2] = c1

    h = h1_ref[...].astype(jnp.float32)
    out_ref[0] = jnp.maximum(ac_ref[:, 0:1] * h + ac_ref[:, 1:2], 0.0)


def _final(h1, s1, gam1, bet1):
    col_spec = pl.BlockSpec((512, 1), lambda b, t: (0, 0))
    return pl.pallas_call(
        _k5_body,
        grid=(B, NB3),
        in_specs=[
            pl.BlockSpec((512, NT3), lambda b, t: (0, b * NB3 + t)),
            pl.BlockSpec((512, 8), lambda b, t: (0, 0)),
            col_spec, col_spec,
        ],
        out_specs=pl.BlockSpec((1, 512, NT3), lambda b, t: (b, 0, t)),
        out_shape=jax.ShapeDtypeStruct((B, 512, N), jnp.float32),
        scratch_shapes=[pltpu.VMEM((512, 2), jnp.float32)],
    )(h1, s1, gam1, bet1)


def kernel(xyz1, xyz2, points1, points2, W0, b0, g0, beta0,
           W1, b1, g1, beta1):
    x1t = jnp.transpose(xyz1, (0, 2, 1))
    x2p = jnp.concatenate(
        [xyz2, jnp.zeros((B, 5, S), xyz2.dtype)], axis=1)

    table = _make_table(points2)
    table2 = table.reshape(B * S * 2, HD)

    idx8a, w4a = _knn_weights(x1t, x2p, 0)
    ga = _gather_rows(table2, idx8a)
    idx8b, w4b = _knn_weights(x1t, x2p, HB)
    gb = _gather_rows(table2, idx8b)

    w0a = W0[:, :D1].astype(jnp.bfloat16)
    w0bp = W0[:, D1:].astype(jnp.bfloat16)
    b0c = b0.reshape(512, 1)

    h0a, s0a = _mlp0(ga.reshape(6, HPTS, HD), w4a, points1,
                     w0a, w0bp, b0c, 0)
    h0, s0b = _mlp0(gb.reshape(6, HPTS, HD), w4b, points1,
                    w0a, w0bp, b0c, HB, h0_prev=h0a)

    h1, s1 = _mlp1(h0, s0a, s0b, g0.reshape(512, 1), beta0.reshape(512, 1),
                   W1.astype(jnp.bfloat16), b1.reshape(512, 1))

    return _final(h1, s1, g1.reshape(512, 1), beta1.reshape(512, 1))


# speedup vs baseline: 1.0662x; 1.0053x over previous
import functools

import jax
import jax.numpy as jnp
from jax import lax
from jax.experimental import pallas as pl
from jax.experimental.pallas import tpu as pltpu
from jax.experimental.pallas import tpu_sc as plsc

B, N, S = 8, 4096, 1024
D1, D2 = 256, 512
HD = D2 // 2
BN_PTS = B * N
HB = B // 2
HPTS = HB * N
NT1 = 1024
NT3 = 256
GW = 128
NB1 = N // NT1
NB3 = N // NT3
INV_CNT = 1.0 / float(BN_PTS)


def _k0_body(p2_ref, t_ref):
    t_ref[...] = jnp.transpose(p2_ref[0])


def _make_table(points2):
    return pl.pallas_call(
        _k0_body,
        grid=(B, 2),
        in_specs=[pl.BlockSpec((1, D2, S // 2), lambda b, t: (b, 0, t))],
        out_specs=pl.BlockSpec((S // 2, D2), lambda b, t: (2 * b + t, 0)),
        out_shape=jax.ShapeDtypeStruct((B * S, D2), jnp.float32),
    )(points2)


def _k1_body(x1_ref, x2_ref, idx_ref, w_ref, *, boff):
    b = pl.program_id(0)
    x1 = x1_ref[0]
    x2 = x2_ref[0]
    x1b = x1.astype(jnp.bfloat16).astype(jnp.float32)
    x2b = x2.astype(jnp.bfloat16).astype(jnp.float32)
    acc = x1b[:, 0:1] * x2b[0:1, :]
    acc = acc + x1b[:, 1:2] * x2b[1:2, :]
    acc = acc + x1b[:, 2:3] * x2b[2:3, :]
    x1sq = jnp.sum(x1 * x1, axis=1, keepdims=True)
    x2sq = jnp.sum(x2 * x2, axis=0, keepdims=True)
    d = (-2.0 * acc + x1sq) + x2sq
    iota = lax.broadcasted_iota(jnp.int32, (NT1, S), 1)
    big = jnp.float32(3.0e38)
    vals, ids = [], []
    work = d
    for _ in range(3):
        m = jnp.min(work, axis=1, keepdims=True)
        sel = jnp.where(work == m, iota, jnp.int32(S))
        ik = jnp.min(sel, axis=1, keepdims=True)
        vals.append(m)
        ids.append(ik)
        work = jnp.where(iota == ik, big, work)
    r = [1.0 / (v + 1e-8) for v in vals]
    rs = r[0] + r[1] + r[2]
    w0 = r[0] / rs
    w1 = r[1] / rs
    w2 = r[2] / rs
    wz = jnp.zeros_like(w0)
    w_ref[0] = jnp.concatenate([w0, w1, w2, wz], axis=1)
    base = (b + boff) * S
    cols = []
    for k in range(3):
        gidx = 2 * (ids[k] + base)
        cols.append(gidx)
        cols.append(gidx + 1)
    zc = jnp.zeros_like(cols[0])
    idx_ref[...] = jnp.transpose(
        jnp.concatenate(cols + [zc, zc], axis=1))


def _knn_weights(x1t, x2p, boff):
    return pl.pallas_call(
        functools.partial(_k1_body, boff=boff),
        grid=(HB, NB1),
        in_specs=[
            pl.BlockSpec((1, NT1, 3), lambda b, t: (b + boff, t, 0)),
            pl.BlockSpec((1, 8, S), lambda b, t: (b + boff, 0, 0)),
        ],
        out_specs=[
            pl.BlockSpec((8, NT1), lambda b, t: (0, b * NB1 + t)),
            pl.BlockSpec((1, NT1, 4), lambda b, t: (b, t, 0)),
        ],
        out_shape=[
            jax.ShapeDtypeStruct((8, HPTS), jnp.int32),
            jax.ShapeDtypeStruct((HB, N, 4), jnp.float32),
        ],
    )(x1t, x2p)


def _gather_rows(table2, idx8):
    nwin = HPTS // GW
    mesh = plsc.VectorSubcoreMesh(core_axis_name="c", subcore_axis_name="s")

    @pl.kernel(
        out_type=jax.ShapeDtypeStruct((6 * HPTS, HD), jnp.float32),
        mesh=mesh,
    )
    def k(tab_hbm, i_hbm, o_hbm):
        def body(i_vmem, o_vmem):
            pltpu.sync_copy(tab_hbm.at[i_vmem.at[0]], o_vmem)

        pltpu.emit_pipeline(
            body,
            grid=(6, nwin),
            in_specs=[pl.BlockSpec((1, GW), lambda q, i: (q, i))],
            out_specs=[pl.BlockSpec((GW, HD), lambda q, i: (q * nwin + i, 0))],
            core_axis_name=("c", "s"),
            dimension_semantics=(pltpu.PARALLEL, pltpu.PARALLEL),
        )(i_hbm, o_hbm)

    return k(table2, idx8)


def _k3_body(g0l_ref, g0h_ref, g1l_ref, g1h_ref, g2l_ref, g2h_ref,
             w_ref, p1_ref, w0a_ref, w0bp_ref, b0_ref, *rest):
    h0_ref, sums_ref = rest[-2], rest[-1]
    p1b = p1_ref[0].astype(jnp.bfloat16)
    h = jnp.dot(w0a_ref[...], p1b, preferred_element_type=jnp.float32)
    w = w_ref[0]
    ilo = w[:, 0:1] * g0l_ref[0]
    ilo = ilo + w[:, 1:2] * g1l_ref[0]
    ilo = ilo + w[:, 2:3] * g2l_ref[0]
    ihi = w[:, 0:1] * g0h_ref[0]
    ihi = ihi + w[:, 1:2] * g1h_ref[0]
    ihi = ihi + w[:, 2:3] * g2h_ref[0]
    interp = jnp.concatenate([ilo, ihi], axis=1).astype(jnp.bfloat16)
    h = h + lax.dot_general(
        w0bp_ref[...], interp,
        (((1,), (1,)), ((), ())), preferred_element_type=jnp.float32)
    h = h + b0_ref[...]
    h0_ref[...] = h.astype(jnp.bfloat16)

    @pl.when(jnp.logical_and(pl.program_id(0) == 0, pl.program_id(1) == 0))
    def _():
        sums_ref[...] = jnp.zeros_like(sums_ref)

    sums_ref[:, 0:1] += jnp.sum(h, axis=1, keepdims=True)
    sums_ref[:, 1:2] += jnp.sum(h * h, axis=1, keepdims=True)


def _mlp0(g6, w4, points1, w0a, w0bp, b0c, boff, h0_prev=None):
    gspec = [pl.BlockSpec((1, NT3, HD),
                          (lambda q: (lambda b, t: (q, b * NB3 + t, 0)))(q))
             for q in range(6)]
    in_specs = gspec + [
        pl.BlockSpec((1, NT3, 4), lambda b, t: (b, t, 0)),
        pl.BlockSpec((1, D1, NT3), lambda b, t: (b + boff, 0, t)),
        pl.BlockSpec((512, D1), lambda b, t: (0, 0)),
        pl.BlockSpec((512, D2), lambda b, t: (0, 0)),
        pl.BlockSpec((512, 1), lambda b, t: (0, 0)),
    ]
    args = [g6, g6, g6, g6, g6, g6, w4, points1, w0a, w0bp, b0c]
    aliases = {}
    if h0_prev is not None:
        in_specs.append(pl.BlockSpec(memory_space=pl.ANY))
        args.append(h0_prev)
        aliases = {len(args) - 1: 0}
    return pl.pallas_call(
        _k3_body,
        grid=(HB, NB3),
        in_specs=in_specs,
        out_specs=[
            pl.BlockSpec((512, NT3), lambda b, t: (0, (b + boff) * NB3 + t)),
            pl.BlockSpec((512, 8), lambda b, t: (0, 0)),
        ],
        out_shape=[
            jax.ShapeDtypeStruct((512, BN_PTS), jnp.bfloat16),
            jax.ShapeDtypeStruct((512, 8), jnp.float32),
        ],
        input_output_aliases=aliases,
    )(*args)


def _bn_ac(sa, sb, gam, bet):
    mean = (sa[:, 0:1] + sb[:, 0:1]) * INV_CNT
    ex2 = (sa[:, 1:2] + sb[:, 1:2]) * INV_CNT
    var = jnp.maximum(ex2 - mean * mean, 0.0)
    a = gam * lax.rsqrt(var + 1e-5)
    c = bet - mean * a
    return a, c


def _k4_body(h0_ref, sa_ref, sb_ref, gam_ref, bet_ref, w1_ref, b1_ref,
             h1_ref, sums_ref, ac_ref):
    @pl.when(pl.program_id(0) == 0)
    def _():
        a0, c0 = _bn_ac(sa_ref[...], sb_ref[...], gam_ref[...], bet_ref[...])
        ac_ref[:, 0:1] = a0
        ac_ref[:, 1:2] = c0

    a = ac_ref[:, 0:1]
    c = ac_ref[:, 1:2]
    h = h0_ref[...].astype(jnp.float32)
    x = jnp.maximum(a * h + c, 0.0)
    h1 = jnp.dot(w1_ref[...], x.astype(jnp.bfloat16),
                 preferred_element_type=jnp.float32)
    h1 = h1 + b1_ref[...]
    h1_ref[...] = h1.astype(jnp.bfloat16)

    @pl.when(pl.program_id(0) == 0)
    def _():
        sums_ref[...] = jnp.zeros_like(sums_ref)

    sums_ref[:, 0:1] += jnp.sum(h1, axis=1, keepdims=True)
    sums_ref[:, 1:2] += jnp.sum(h1 * h1, axis=1, keepdims=True)


def _mlp1(h0, s0a, s0b, gam0, bet0, w1, b1c):
    stat_spec = pl.BlockSpec((512, 8), lambda i: (0, 0))
    col_spec = pl.BlockSpec((512, 1), lambda i: (0, 0))
    return pl.pallas_call(
        _k4_body,
        grid=(BN_PTS // NT3,),
        in_specs=[
            pl.BlockSpec((512, NT3), lambda i: (0, i)),
            stat_spec, stat_spec, col_spec, col_spec,
            pl.BlockSpec((512, 512), lambda i: (0, 0)),
            col_spec,
        ],
        out_specs=[
            pl.BlockSpec((512, NT3), lambda i: (0, i)),
            pl.BlockSpec((512, 8), lambda i: (0, 0)),
        ],
        out_shape=[
            jax.ShapeDtypeStruct((512, BN_PTS), jnp.bfloat16),
            jax.ShapeDtypeStruct((512, 8), jnp.float32),
        ],
        scratch_shapes=[pltpu.VMEM((512, 2), jnp.float32)],
    )(h0, s0a, s0b, gam0, bet0, w1, b1c)


def _k5_body(h1_ref, s1_ref, gam_ref, bet_ref, out_ref, ac_ref):
    first = jnp.logical_and(pl.program_id(0) == 0, pl.program_id(1) == 0)

    @pl.when(first)
    def _():
        z = jnp.zeros_like(s1_ref[...])
        a1, c1 = _bn_ac(s1_ref[...], z, gam_ref[...], bet_ref[...])
        ac_ref[:, 0:1] = a1
        ac_ref[:, 1:2] = c1

    h = h1_ref[...].astype(jnp.float32)
    out_ref[0] = jnp.maximum(ac_ref[:, 0:1] * h + ac_ref[:, 1:2], 0.0)


def _final(h1, s1, gam1, bet1):
    col_spec = pl.BlockSpec((512, 1), lambda b, t: (0, 0))
    return pl.pallas_call(
        _k5_body,
        grid=(B, NB3),
        in_specs=[
            pl.BlockSpec((512, NT3), lambda b, t: (0, b * NB3 + t)),
            pl.BlockSpec((512, 8), lambda b, t: (0, 0)),
            col_spec, col_spec,
        ],
        out_specs=pl.BlockSpec((1, 512, NT3), lambda b, t: (b, 0, t)),
        out_shape=jax.ShapeDtypeStruct((B, 512, N), jnp.float32),
        scratch_shapes=[pltpu.VMEM((512, 2), jnp.float32)],
    )(h1, s1, gam1, bet1)


def kernel(xyz1, xyz2, points1, points2, W0, b0, g0, beta0,
           W1, b1, g1, beta1):
    x1t = jnp.transpose(xyz1, (0, 2, 1))
    x2p = jnp.concatenate(
        [xyz2, jnp.zeros((B, 5, S), xyz2.dtype)], axis=1)

    table = _make_table(points2)
    table2 = table.reshape(B * S * 2, HD)

    idx8a, w4a = _knn_weights(x1t, x2p, 0)
    ga = _gather_rows(table2, idx8a)
    idx8b, w4b = _knn_weights(x1t, x2p, HB)
    gb = _gather_rows(table2, idx8b)

    w0a = W0[:, :D1].astype(jnp.bfloat16)
    w0bp = W0[:, D1:].astype(jnp.bfloat16)
    b0c = b0.reshape(512, 1)

    h0a, s0a = _mlp0(ga.reshape(6, HPTS, HD), w4a, points1,
                     w0a, w0bp, b0c, 0)
    h0, s0b = _mlp0(gb.reshape(6, HPTS, HD), w4b, points1,
                    w0a, w0bp, b0c, HB, h0_prev=h0a)

    h1, s1 = _mlp1(h0, s0a, s0b, g0.reshape(512, 1), beta0.reshape(512, 1),
                   W1.astype(jnp.bfloat16), b1.reshape(512, 1))

    return _final(h1, s1, g1.reshape(512, 1), beta1.reshape(512, 1))


# speedup vs baseline: 1.3071x; 1.2259x over previous
import functools

import jax
import jax.numpy as jnp
from jax import lax
from jax.experimental import pallas as pl
from jax.experimental.pallas import tpu as pltpu
from jax.experimental.pallas import tpu_sc as plsc

B, N, S = 8, 4096, 1024
D1, D2 = 256, 512
HD = D2 // 2
BN_PTS = B * N
HB = B // 2
HPTS = HB * N
NT1 = 1024
NT3 = 512
GW = 128
NB1 = N // NT1
NB3 = N // NT3
INV_CNT = 1.0 / float(BN_PTS)


def _k0_body(p2_ref, t_ref):
    t_ref[...] = jnp.transpose(p2_ref[0])


def _make_table(points2):
    return pl.pallas_call(
        _k0_body,
        grid=(B, 2),
        in_specs=[pl.BlockSpec((1, D2, S // 2), lambda b, t: (b, 0, t))],
        out_specs=pl.BlockSpec((S // 2, D2), lambda b, t: (2 * b + t, 0)),
        out_shape=jax.ShapeDtypeStruct((B * S, D2), jnp.float32),
    )(points2)


def _k1_body(x1_ref, x2_ref, idx_ref, w_ref, *, boff):
    b = pl.program_id(0)
    x1 = x1_ref[0]
    x2 = x2_ref[0]
    x1b = x1.astype(jnp.bfloat16).astype(jnp.float32)
    x2b = x2.astype(jnp.bfloat16).astype(jnp.float32)
    acc = x1b[:, 0:1] * x2b[0:1, :]
    acc = acc + x1b[:, 1:2] * x2b[1:2, :]
    acc = acc + x1b[:, 2:3] * x2b[2:3, :]
    x1sq = jnp.sum(x1 * x1, axis=1, keepdims=True)
    x2sq = jnp.sum(x2 * x2, axis=0, keepdims=True)
    d = (-2.0 * acc + x1sq) + x2sq
    iota = lax.broadcasted_iota(jnp.int32, (NT1, S), 1)
    big = jnp.float32(3.0e38)
    vals, ids = [], []
    work = d
    for _ in range(3):
        m = jnp.min(work, axis=1, keepdims=True)
        sel = jnp.where(work == m, iota, jnp.int32(S))
        ik = jnp.min(sel, axis=1, keepdims=True)
        vals.append(m)
        ids.append(ik)
        work = jnp.where(iota == ik, big, work)
    r = [1.0 / (v + 1e-8) for v in vals]
    rs = r[0] + r[1] + r[2]
    w0 = r[0] / rs
    w1 = r[1] / rs
    w2 = r[2] / rs
    wz = jnp.zeros_like(w0)
    w_ref[0] = jnp.concatenate([w0, w1, w2, wz], axis=1)
    base = (b + boff) * S
    cols = []
    for k in range(3):
        gidx = 2 * (ids[k] + base)
        cols.append(gidx)
        cols.append(gidx + 1)
    zc = jnp.zeros_like(cols[0])
    idx_ref[...] = jnp.transpose(
        jnp.concatenate(cols + [zc, zc], axis=1))


def _knn_weights(x1t, x2p, boff):
    return pl.pallas_call(
        functools.partial(_k1_body, boff=boff),
        grid=(HB, NB1),
        in_specs=[
            pl.BlockSpec((1, NT1, 3), lambda b, t: (b + boff, t, 0)),
            pl.BlockSpec((1, 8, S), lambda b, t: (b + boff, 0, 0)),
        ],
        out_specs=[
            pl.BlockSpec((8, NT1), lambda b, t: (0, b * NB1 + t)),
            pl.BlockSpec((1, NT1, 4), lambda b, t: (b, t, 0)),
        ],
        out_shape=[
            jax.ShapeDtypeStruct((8, HPTS), jnp.int32),
            jax.ShapeDtypeStruct((HB, N, 4), jnp.float32),
        ],
    )(x1t, x2p)


def _gather_rows(table2, idx8):
    nwin = HPTS // GW
    mesh = plsc.VectorSubcoreMesh(core_axis_name="c", subcore_axis_name="s")

    @pl.kernel(
        out_type=jax.ShapeDtypeStruct((6 * HPTS, HD), jnp.float32),
        mesh=mesh,
    )
    def k(tab_hbm, i_hbm, o_hbm):
        def body(i_vmem, o_vmem):
            pltpu.sync_copy(tab_hbm.at[i_vmem.at[0]], o_vmem)

        pltpu.emit_pipeline(
            body,
            grid=(6, nwin),
            in_specs=[pl.BlockSpec((1, GW), lambda q, i: (q, i))],
            out_specs=[pl.BlockSpec((GW, HD), lambda q, i: (q * nwin + i, 0))],
            core_axis_name=("c", "s"),
            dimension_semantics=(pltpu.PARALLEL, pltpu.PARALLEL),
        )(i_hbm, o_hbm)

    return k(table2, idx8)


def _k3_body(g0l_ref, g0h_ref, g1l_ref, g1h_ref, g2l_ref, g2h_ref,
             w_ref, p1_ref, w0a_ref, w0bp_ref, b0_ref, *rest):
    h0_ref, sums_ref = rest[-2], rest[-1]
    p1b = p1_ref[0].astype(jnp.bfloat16)
    h = jnp.dot(w0a_ref[...], p1b, preferred_element_type=jnp.float32)
    w = w_ref[0]
    ilo = w[:, 0:1] * g0l_ref[0]
    ilo = ilo + w[:, 1:2] * g1l_ref[0]
    ilo = ilo + w[:, 2:3] * g2l_ref[0]
    ihi = w[:, 0:1] * g0h_ref[0]
    ihi = ihi + w[:, 1:2] * g1h_ref[0]
    ihi = ihi + w[:, 2:3] * g2h_ref[0]
    interp = jnp.concatenate([ilo, ihi], axis=1).astype(jnp.bfloat16)
    h = h + lax.dot_general(
        w0bp_ref[...], interp,
        (((1,), (1,)), ((), ())), preferred_element_type=jnp.float32)
    h = h + b0_ref[...]
    h0_ref[...] = h.astype(jnp.bfloat16)

    @pl.when(jnp.logical_and(pl.program_id(0) == 0, pl.program_id(1) == 0))
    def _():
        sums_ref[...] = jnp.zeros_like(sums_ref)

    sums_ref[:, 0:1] += jnp.sum(h, axis=1, keepdims=True)
    sums_ref[:, 1:2] += jnp.sum(h * h, axis=1, keepdims=True)


def _mlp0(g6, w4, points1, w0a, w0bp, b0c, boff, h0_prev=None):
    gspec = [pl.BlockSpec((1, NT3, HD),
                          (lambda q: (lambda b, t: (q, b * NB3 + t, 0)))(q))
             for q in range(6)]
    in_specs = gspec + [
        pl.BlockSpec((1, NT3, 4), lambda b, t: (b, t, 0)),
        pl.BlockSpec((1, D1, NT3), lambda b, t: (b + boff, 0, t)),
        pl.BlockSpec((512, D1), lambda b, t: (0, 0)),
        pl.BlockSpec((512, D2), lambda b, t: (0, 0)),
        pl.BlockSpec((512, 1), lambda b, t: (0, 0)),
    ]
    args = [g6, g6, g6, g6, g6, g6, w4, points1, w0a, w0bp, b0c]
    aliases = {}
    if h0_prev is not None:
        in_specs.append(pl.BlockSpec(memory_space=pl.ANY))
        args.append(h0_prev)
        aliases = {len(args) - 1: 0}
    return pl.pallas_call(
        _k3_body,
        grid=(HB, NB3),
        in_specs=in_specs,
        out_specs=[
            pl.BlockSpec((512, NT3), lambda b, t: (0, (b + boff) * NB3 + t)),
            pl.BlockSpec((512, 8), lambda b, t: (0, 0)),
        ],
        out_shape=[
            jax.ShapeDtypeStruct((512, BN_PTS), jnp.bfloat16),
            jax.ShapeDtypeStruct((512, 8), jnp.float32),
        ],
        input_output_aliases=aliases,
    )(*args)


def _bn_ac(sa, sb, gam, bet):
    mean = (sa[:, 0:1] + sb[:, 0:1]) * INV_CNT
    ex2 = (sa[:, 1:2] + sb[:, 1:2]) * INV_CNT
    var = jnp.maximum(ex2 - mean * mean, 0.0)
    a = gam * lax.rsqrt(var + 1e-5)
    c = bet - mean * a
    return a, c


def _k4_body(h0_ref, sa_ref, sb_ref, gam_ref, bet_ref, w1_ref, b1_ref,
             h1_ref, sums_ref, ac_ref):
    @pl.when(pl.program_id(0) == 0)
    def _():
        a0, c0 = _bn_ac(sa_ref[...], sb_ref[...], gam_ref[...], bet_ref[...])
        ac_ref[:, 0:1] = a0
        ac_ref[:, 1:2] = c0

    a = ac_ref[:, 0:1]
    c = ac_ref[:, 1:2]
    h = h0_ref[...].astype(jnp.float32)
    x = jnp.maximum(a * h + c, 0.0)
    h1 = jnp.dot(w1_ref[...], x.astype(jnp.bfloat16),
                 preferred_element_type=jnp.float32)
    h1 = h1 + b1_ref[...]
    h1_ref[...] = h1.astype(jnp.bfloat16)

    @pl.when(pl.program_id(0) == 0)
    def _():
        sums_ref[...] = jnp.zeros_like(sums_ref)

    sums_ref[:, 0:1] += jnp.sum(h1, axis=1, keepdims=True)
    sums_ref[:, 1:2] += jnp.sum(h1 * h1, axis=1, keepdims=True)


def _mlp1(h0, s0a, s0b, gam0, bet0, w1, b1c):
    stat_spec = pl.BlockSpec((512, 8), lambda i: (0, 0))
    col_spec = pl.BlockSpec((512, 1), lambda i: (0, 0))
    return pl.pallas_call(
        _k4_body,
        grid=(BN_PTS // NT3,),
        in_specs=[
            pl.BlockSpec((512, NT3), lambda i: (0, i)),
            stat_spec, stat_spec, col_spec, col_spec,
            pl.BlockSpec((512, 512), lambda i: (0, 0)),
            col_spec,
        ],
        out_specs=[
            pl.BlockSpec((512, NT3), lambda i: (0, i)),
            pl.BlockSpec((512, 8), lambda i: (0, 0)),
        ],
        out_shape=[
            jax.ShapeDtypeStruct((512, BN_PTS), jnp.bfloat16),
            jax.ShapeDtypeStruct((512, 8), jnp.float32),
        ],
        scratch_shapes=[pltpu.VMEM((512, 2), jnp.float32)],
    )(h0, s0a, s0b, gam0, bet0, w1, b1c)


def _k5_body(h1_ref, s1_ref, gam_ref, bet_ref, out_ref, ac_ref):
    first = jnp.logical_and(pl.program_id(0) == 0, pl.program_id(1) == 0)

    @pl.when(first)
    def _():
        z = jnp.zeros_like(s1_ref[...])
        a1, c1 = _bn_ac(s1_ref[...], z, gam_ref[...], bet_ref[...])
        ac_ref[:, 0:1] = a1
        ac_ref[:, 1:2] = c1

    h = h1_ref[...].astype(jnp.float32)
    out_ref[0] = jnp.maximum(ac_ref[:, 0:1] * h + ac_ref[:, 1:2], 0.0)


def _final(h1, s1, gam1, bet1):
    col_spec = pl.BlockSpec((512, 1), lambda b, t: (0, 0))
    return pl.pallas_call(
        _k5_body,
        grid=(B, NB3),
        in_specs=[
            pl.BlockSpec((512, NT3), lambda b, t: (0, b * NB3 + t)),
            pl.BlockSpec((512, 8), lambda b, t: (0, 0)),
            col_spec, col_spec,
        ],
        out_specs=pl.BlockSpec((1, 512, NT3), lambda b, t: (b, 0, t)),
        out_shape=jax.ShapeDtypeStruct((B, 512, N), jnp.float32),
        scratch_shapes=[pltpu.VMEM((512, 2), jnp.float32)],
    )(h1, s1, gam1, bet1)


def kernel(xyz1, xyz2, points1, points2, W0, b0, g0, beta0,
           W1, b1, g1, beta1):
    x1t = jnp.transpose(xyz1, (0, 2, 1))
    x2p = jnp.concatenate(
        [xyz2, jnp.zeros((B, 5, S), xyz2.dtype)], axis=1)

    table = _make_table(points2)
    table2 = table.reshape(B * S * 2, HD)

    idx8a, w4a = _knn_weights(x1t, x2p, 0)
    ga = _gather_rows(table2, idx8a)
    idx8b, w4b = _knn_weights(x1t, x2p, HB)
    gb = _gather_rows(table2, idx8b)

    w0a = W0[:, :D1].astype(jnp.bfloat16)
    w0bp = W0[:, D1:].astype(jnp.bfloat16)
    b0c = b0.reshape(512, 1)

    h0a, s0a = _mlp0(ga.reshape(6, HPTS, HD), w4a, points1,
                     w0a, w0bp, b0c, 0)
    h0, s0b = _mlp0(gb.reshape(6, HPTS, HD), w4b, points1,
                    w0a, w0bp, b0c, HB, h0_prev=h0a)

    h1, s1 = _mlp1(h0, s0a, s0b, g0.reshape(512, 1), beta0.reshape(512, 1),
                   W1.astype(jnp.bfloat16), b1.reshape(512, 1))

    return _final(h1, s1, g1.reshape(512, 1), beta1.reshape(512, 1))


# speedup vs baseline: 1.4513x; 1.1103x over previous
import functools

import jax
import jax.numpy as jnp
from jax import lax
from jax.experimental import pallas as pl
from jax.experimental.pallas import tpu as pltpu
from jax.experimental.pallas import tpu_sc as plsc

B, N, S = 8, 4096, 1024
D1, D2 = 256, 512
HD = D2 // 2
BN_PTS = B * N
HB = B // 2
HPTS = HB * N
NT1 = 1024
NT3 = 1024
GW = 128
NB1 = N // NT1
NB3 = N // NT3
INV_CNT = 1.0 / float(BN_PTS)


def _k0_body(p2_ref, t_ref):
    t_ref[...] = jnp.transpose(p2_ref[0])


def _make_table(points2):
    return pl.pallas_call(
        _k0_body,
        grid=(B, 2),
        in_specs=[pl.BlockSpec((1, D2, S // 2), lambda b, t: (b, 0, t))],
        out_specs=pl.BlockSpec((S // 2, D2), lambda b, t: (2 * b + t, 0)),
        out_shape=jax.ShapeDtypeStruct((B * S, D2), jnp.float32),
    )(points2)


def _k1_body(x1_ref, x2_ref, idx_ref, w_ref, *, boff):
    b = pl.program_id(0)
    x1 = x1_ref[0]
    x2 = x2_ref[0]
    x1b = x1.astype(jnp.bfloat16).astype(jnp.float32)
    x2b = x2.astype(jnp.bfloat16).astype(jnp.float32)
    acc = x1b[:, 0:1] * x2b[0:1, :]
    acc = acc + x1b[:, 1:2] * x2b[1:2, :]
    acc = acc + x1b[:, 2:3] * x2b[2:3, :]
    x1sq = jnp.sum(x1 * x1, axis=1, keepdims=True)
    x2sq = jnp.sum(x2 * x2, axis=0, keepdims=True)
    d = (-2.0 * acc + x1sq) + x2sq
    iota = lax.broadcasted_iota(jnp.int32, (NT1, S), 1)
    big = jnp.float32(3.0e38)
    vals, ids = [], []
    work = d
    for _ in range(3):
        m = jnp.min(work, axis=1, keepdims=True)
        sel = jnp.where(work == m, iota, jnp.int32(S))
        ik = jnp.min(sel, axis=1, keepdims=True)
        vals.append(m)
        ids.append(ik)
        work = jnp.where(iota == ik, big, work)
    r = [1.0 / (v + 1e-8) for v in vals]
    rs = r[0] + r[1] + r[2]
    w0 = r[0] / rs
    w1 = r[1] / rs
    w2 = r[2] / rs
    wz = jnp.zeros_like(w0)
    w_ref[0] = jnp.concatenate([w0, w1, w2, wz], axis=1)
    base = (b + boff) * S
    cols = []
    for k in range(3):
        gidx = 2 * (ids[k] + base)
        cols.append(gidx)
        cols.append(gidx + 1)
    zc = jnp.zeros_like(cols[0])
    idx_ref[...] = jnp.transpose(
        jnp.concatenate(cols + [zc, zc], axis=1))


def _knn_weights(x1t, x2p, boff):
    return pl.pallas_call(
        functools.partial(_k1_body, boff=boff),
        grid=(HB, NB1),
        in_specs=[
            pl.BlockSpec((1, NT1, 3), lambda b, t: (b + boff, t, 0)),
            pl.BlockSpec((1, 8, S), lambda b, t: (b + boff, 0, 0)),
        ],
        out_specs=[
            pl.BlockSpec((8, NT1), lambda b, t: (0, b * NB1 + t)),
            pl.BlockSpec((1, NT1, 4), lambda b, t: (b, t, 0)),
        ],
        out_shape=[
            jax.ShapeDtypeStruct((8, HPTS), jnp.int32),
            jax.ShapeDtypeStruct((HB, N, 4), jnp.float32),
        ],
    )(x1t, x2p)


def _gather_rows(table2, idx8):
    nwin = HPTS // GW
    mesh = plsc.VectorSubcoreMesh(core_axis_name="c", subcore_axis_name="s")

    @pl.kernel(
        out_type=jax.ShapeDtypeStruct((6 * HPTS, HD), jnp.float32),
        mesh=mesh,
    )
    def k(tab_hbm, i_hbm, o_hbm):
        def body(i_vmem, o_vmem):
            pltpu.sync_copy(tab_hbm.at[i_vmem.at[0]], o_vmem)

        pltpu.emit_pipeline(
            body,
            grid=(6, nwin),
            in_specs=[pl.BlockSpec((1, GW), lambda q, i: (q, i))],
            out_specs=[pl.BlockSpec((GW, HD), lambda q, i: (q * nwin + i, 0))],
            core_axis_name=("c", "s"),
            dimension_semantics=(pltpu.PARALLEL, pltpu.PARALLEL),
        )(i_hbm, o_hbm)

    return k(table2, idx8)


def _k3_body(g0l_ref, g0h_ref, g1l_ref, g1h_ref, g2l_ref, g2h_ref,
             w_ref, p1_ref, w0a_ref, w0bp_ref, b0_ref, *rest):
    h0_ref, sums_ref = rest[-2], rest[-1]
    p1b = p1_ref[0].astype(jnp.bfloat16)
    h = jnp.dot(w0a_ref[...], p1b, preferred_element_type=jnp.float32)
    w = w_ref[0]
    ilo = w[:, 0:1] * g0l_ref[0]
    ilo = ilo + w[:, 1:2] * g1l_ref[0]
    ilo = ilo + w[:, 2:3] * g2l_ref[0]
    ihi = w[:, 0:1] * g0h_ref[0]
    ihi = ihi + w[:, 1:2] * g1h_ref[0]
    ihi = ihi + w[:, 2:3] * g2h_ref[0]
    interp = jnp.concatenate([ilo, ihi], axis=1).astype(jnp.bfloat16)
    h = h + lax.dot_general(
        w0bp_ref[...], interp,
        (((1,), (1,)), ((), ())), preferred_element_type=jnp.float32)
    h = h + b0_ref[...]
    h0_ref[...] = h.astype(jnp.bfloat16)

    @pl.when(jnp.logical_and(pl.program_id(0) == 0, pl.program_id(1) == 0))
    def _():
        sums_ref[...] = jnp.zeros_like(sums_ref)

    sums_ref[:, 0:1] += jnp.sum(h, axis=1, keepdims=True)
    sums_ref[:, 1:2] += jnp.sum(h * h, axis=1, keepdims=True)


def _mlp0(g6, w4, points1, w0a, w0bp, b0c, boff, h0_prev=None):
    gspec = [pl.BlockSpec((1, NT3, HD),
                          (lambda q: (lambda b, t: (q, b * NB3 + t, 0)))(q))
             for q in range(6)]
    in_specs = gspec + [
        pl.BlockSpec((1, NT3, 4), lambda b, t: (b, t, 0)),
        pl.BlockSpec((1, D1, NT3), lambda b, t: (b + boff, 0, t)),
        pl.BlockSpec((512, D1), lambda b, t: (0, 0)),
        pl.BlockSpec((512, D2), lambda b, t: (0, 0)),
        pl.BlockSpec((512, 1), lambda b, t: (0, 0)),
    ]
    args = [g6, g6, g6, g6, g6, g6, w4, points1, w0a, w0bp, b0c]
    aliases = {}
    if h0_prev is not None:
        in_specs.append(pl.BlockSpec(memory_space=pl.ANY))
        args.append(h0_prev)
        aliases = {len(args) - 1: 0}
    return pl.pallas_call(
        _k3_body,
        grid=(HB, NB3),
        in_specs=in_specs,
        out_specs=[
            pl.BlockSpec((512, NT3), lambda b, t: (0, (b + boff) * NB3 + t)),
            pl.BlockSpec((512, 8), lambda b, t: (0, 0)),
        ],
        out_shape=[
            jax.ShapeDtypeStruct((512, BN_PTS), jnp.bfloat16),
            jax.ShapeDtypeStruct((512, 8), jnp.float32),
        ],
        input_output_aliases=aliases,
    )(*args)


def _bn_ac(sa, sb, gam, bet):
    mean = (sa[:, 0:1] + sb[:, 0:1]) * INV_CNT
    ex2 = (sa[:, 1:2] + sb[:, 1:2]) * INV_CNT
    var = jnp.maximum(ex2 - mean * mean, 0.0)
    a = gam * lax.rsqrt(var + 1e-5)
    c = bet - mean * a
    return a, c


def _k4_body(h0_ref, sa_ref, sb_ref, gam_ref, bet_ref, w1_ref, b1_ref,
             h1_ref, sums_ref, ac_ref):
    @pl.when(pl.program_id(0) == 0)
    def _():
        a0, c0 = _bn_ac(sa_ref[...], sb_ref[...], gam_ref[...], bet_ref[...])
        ac_ref[:, 0:1] = a0
        ac_ref[:, 1:2] = c0

    a = ac_ref[:, 0:1]
    c = ac_ref[:, 1:2]
    h = h0_ref[...].astype(jnp.float32)
    x = jnp.maximum(a * h + c, 0.0)
    h1 = jnp.dot(w1_ref[...], x.astype(jnp.bfloat16),
                 preferred_element_type=jnp.float32)
    h1 = h1 + b1_ref[...]
    h1_ref[...] = h1.astype(jnp.bfloat16)

    @pl.when(pl.program_id(0) == 0)
    def _():
        sums_ref[...] = jnp.zeros_like(sums_ref)

    sums_ref[:, 0:1] += jnp.sum(h1, axis=1, keepdims=True)
    sums_ref[:, 1:2] += jnp.sum(h1 * h1, axis=1, keepdims=True)


def _mlp1(h0, s0a, s0b, gam0, bet0, w1, b1c):
    stat_spec = pl.BlockSpec((512, 8), lambda i: (0, 0))
    col_spec = pl.BlockSpec((512, 1), lambda i: (0, 0))
    return pl.pallas_call(
        _k4_body,
        grid=(BN_PTS // NT3,),
        in_specs=[
            pl.BlockSpec((512, NT3), lambda i: (0, i)),
            stat_spec, stat_spec, col_spec, col_spec,
            pl.BlockSpec((512, 512), lambda i: (0, 0)),
            col_spec,
        ],
        out_specs=[
            pl.BlockSpec((512, NT3), lambda i: (0, i)),
            pl.BlockSpec((512, 8), lambda i: (0, 0)),
        ],
        out_shape=[
            jax.ShapeDtypeStruct((512, BN_PTS), jnp.bfloat16),
            jax.ShapeDtypeStruct((512, 8), jnp.float32),
        ],
        scratch_shapes=[pltpu.VMEM((512, 2), jnp.float32)],
    )(h0, s0a, s0b, gam0, bet0, w1, b1c)


def _k5_body(h1_ref, s1_ref, gam_ref, bet_ref, out_ref, ac_ref):
    first = jnp.logical_and(pl.program_id(0) == 0, pl.program_id(1) == 0)

    @pl.when(first)
    def _():
        z = jnp.zeros_like(s1_ref[...])
        a1, c1 = _bn_ac(s1_ref[...], z, gam_ref[...], bet_ref[...])
        ac_ref[:, 0:1] = a1
        ac_ref[:, 1:2] = c1

    h = h1_ref[...].astype(jnp.float32)
    out_ref[0] = jnp.maximum(ac_ref[:, 0:1] * h + ac_ref[:, 1:2], 0.0)


def _final(h1, s1, gam1, bet1):
    col_spec = pl.BlockSpec((512, 1), lambda b, t: (0, 0))
    return pl.pallas_call(
        _k5_body,
        grid=(B, NB3),
        in_specs=[
            pl.BlockSpec((512, NT3), lambda b, t: (0, b * NB3 + t)),
            pl.BlockSpec((512, 8), lambda b, t: (0, 0)),
            col_spec, col_spec,
        ],
        out_specs=pl.BlockSpec((1, 512, NT3), lambda b, t: (b, 0, t)),
        out_shape=jax.ShapeDtypeStruct((B, 512, N), jnp.float32),
        scratch_shapes=[pltpu.VMEM((512, 2), jnp.float32)],
    )(h1, s1, gam1, bet1)


def kernel(xyz1, xyz2, points1, points2, W0, b0, g0, beta0,
           W1, b1, g1, beta1):
    x1t = jnp.transpose(xyz1, (0, 2, 1))
    x2p = jnp.concatenate(
        [xyz2, jnp.zeros((B, 5, S), xyz2.dtype)], axis=1)

    table = _make_table(points2)
    table2 = table.reshape(B * S * 2, HD)

    idx8a, w4a = _knn_weights(x1t, x2p, 0)
    ga = _gather_rows(table2, idx8a)
    idx8b, w4b = _knn_weights(x1t, x2p, HB)
    gb = _gather_rows(table2, idx8b)

    w0a = W0[:, :D1].astype(jnp.bfloat16)
    w0bp = W0[:, D1:].astype(jnp.bfloat16)
    b0c = b0.reshape(512, 1)

    h0a, s0a = _mlp0(ga.reshape(6, HPTS, HD), w4a, points1,
                     w0a, w0bp, b0c, 0)
    h0, s0b = _mlp0(gb.reshape(6, HPTS, HD), w4b, points1,
                    w0a, w0bp, b0c, HB, h0_prev=h0a)

    h1, s1 = _mlp1(h0, s0a, s0b, g0.reshape(512, 1), beta0.reshape(512, 1),
                   W1.astype(jnp.bfloat16), b1.reshape(512, 1))

    return _final(h1, s1, g1.reshape(512, 1), beta1.reshape(512, 1))


# speedup vs baseline: 1.5180x; 1.0459x over previous
import functools

import jax
import jax.numpy as jnp
from jax import lax
from jax.experimental import pallas as pl
from jax.experimental.pallas import tpu as pltpu
from jax.experimental.pallas import tpu_sc as plsc

B, N, S = 8, 4096, 1024
D1, D2 = 256, 512
HD = D2 // 2
BN_PTS = B * N
HB = B // 2
HPTS = HB * N
NT1 = 1024
NT3 = 2048
GW = 128
NB1 = N // NT1
NB3 = N // NT3
INV_CNT = 1.0 / float(BN_PTS)


def _k0_body(p2_ref, t_ref):
    t_ref[...] = jnp.transpose(p2_ref[0])


def _make_table(points2):
    return pl.pallas_call(
        _k0_body,
        grid=(B, 2),
        in_specs=[pl.BlockSpec((1, D2, S // 2), lambda b, t: (b, 0, t))],
        out_specs=pl.BlockSpec((S // 2, D2), lambda b, t: (2 * b + t, 0)),
        out_shape=jax.ShapeDtypeStruct((B * S, D2), jnp.float32),
    )(points2)


def _k1_body(x1_ref, x2_ref, idx_ref, w_ref, *, boff):
    b = pl.program_id(0)
    x1 = x1_ref[0]
    x2 = x2_ref[0]
    x1b = x1.astype(jnp.bfloat16).astype(jnp.float32)
    x2b = x2.astype(jnp.bfloat16).astype(jnp.float32)
    acc = x1b[:, 0:1] * x2b[0:1, :]
    acc = acc + x1b[:, 1:2] * x2b[1:2, :]
    acc = acc + x1b[:, 2:3] * x2b[2:3, :]
    x1sq = jnp.sum(x1 * x1, axis=1, keepdims=True)
    x2sq = jnp.sum(x2 * x2, axis=0, keepdims=True)
    d = (-2.0 * acc + x1sq) + x2sq
    iota = lax.broadcasted_iota(jnp.int32, (NT1, S), 1)
    big = jnp.float32(3.0e38)
    vals, ids = [], []
    work = d
    for _ in range(3):
        m = jnp.min(work, axis=1, keepdims=True)
        sel = jnp.where(work == m, iota, jnp.int32(S))
        ik = jnp.min(sel, axis=1, keepdims=True)
        vals.append(m)
        ids.append(ik)
        work = jnp.where(iota == ik, big, work)
    r = [1.0 / (v + 1e-8) for v in vals]
    rs = r[0] + r[1] + r[2]
    w0 = r[0] / rs
    w1 = r[1] / rs
    w2 = r[2] / rs
    wz = jnp.zeros_like(w0)
    w_ref[0] = jnp.concatenate([w0, w1, w2, wz], axis=1)
    base = (b + boff) * S
    cols = []
    for k in range(3):
        gidx = 2 * (ids[k] + base)
        cols.append(gidx)
        cols.append(gidx + 1)
    zc = jnp.zeros_like(cols[0])
    idx_ref[...] = jnp.transpose(
        jnp.concatenate(cols + [zc, zc], axis=1))


def _knn_weights(x1t, x2p, boff):
    return pl.pallas_call(
        functools.partial(_k1_body, boff=boff),
        grid=(HB, NB1),
        in_specs=[
            pl.BlockSpec((1, NT1, 3), lambda b, t: (b + boff, t, 0)),
            pl.BlockSpec((1, 8, S), lambda b, t: (b + boff, 0, 0)),
        ],
        out_specs=[
            pl.BlockSpec((8, NT1), lambda b, t: (0, b * NB1 + t)),
            pl.BlockSpec((1, NT1, 4), lambda b, t: (b, t, 0)),
        ],
        out_shape=[
            jax.ShapeDtypeStruct((8, HPTS), jnp.int32),
            jax.ShapeDtypeStruct((HB, N, 4), jnp.float32),
        ],
    )(x1t, x2p)


def _gather_rows(table2, idx8):
    nwin = HPTS // GW
    mesh = plsc.VectorSubcoreMesh(core_axis_name="c", subcore_axis_name="s")

    @pl.kernel(
        out_type=jax.ShapeDtypeStruct((6 * HPTS, HD), jnp.float32),
        mesh=mesh,
    )
    def k(tab_hbm, i_hbm, o_hbm):
        def body(i_vmem, o_vmem):
            pltpu.sync_copy(tab_hbm.at[i_vmem.at[0]], o_vmem)

        pltpu.emit_pipeline(
            body,
            grid=(6, nwin),
            in_specs=[pl.BlockSpec((1, GW), lambda q, i: (q, i))],
            out_specs=[pl.BlockSpec((GW, HD), lambda q, i: (q * nwin + i, 0))],
            core_axis_name=("c", "s"),
            dimension_semantics=(pltpu.PARALLEL, pltpu.PARALLEL),
        )(i_hbm, o_hbm)

    return k(table2, idx8)


def _k3_body(g0l_ref, g0h_ref, g1l_ref, g1h_ref, g2l_ref, g2h_ref,
             w_ref, p1_ref, w0a_ref, w0bp_ref, b0_ref, *rest):
    h0_ref, sums_ref = rest[-2], rest[-1]
    p1b = p1_ref[0].astype(jnp.bfloat16)
    h = jnp.dot(w0a_ref[...], p1b, preferred_element_type=jnp.float32)
    w = w_ref[0]
    ilo = w[:, 0:1] * g0l_ref[0]
    ilo = ilo + w[:, 1:2] * g1l_ref[0]
    ilo = ilo + w[:, 2:3] * g2l_ref[0]
    ihi = w[:, 0:1] * g0h_ref[0]
    ihi = ihi + w[:, 1:2] * g1h_ref[0]
    ihi = ihi + w[:, 2:3] * g2h_ref[0]
    interp = jnp.concatenate([ilo, ihi], axis=1).astype(jnp.bfloat16)
    h = h + lax.dot_general(
        w0bp_ref[...], interp,
        (((1,), (1,)), ((), ())), preferred_element_type=jnp.float32)
    h = h + b0_ref[...]
    h0_ref[...] = h.astype(jnp.bfloat16)

    @pl.when(jnp.logical_and(pl.program_id(0) == 0, pl.program_id(1) == 0))
    def _():
        sums_ref[...] = jnp.zeros_like(sums_ref)

    sums_ref[:, 0:1] += jnp.sum(h, axis=1, keepdims=True)
    sums_ref[:, 1:2] += jnp.sum(h * h, axis=1, keepdims=True)


def _mlp0(g6, w4, points1, w0a, w0bp, b0c, boff, h0_prev=None):
    gspec = [pl.BlockSpec((1, NT3, HD),
                          (lambda q: (lambda b, t: (q, b * NB3 + t, 0)))(q))
             for q in range(6)]
    in_specs = gspec + [
        pl.BlockSpec((1, NT3, 4), lambda b, t: (b, t, 0)),
        pl.BlockSpec((1, D1, NT3), lambda b, t: (b + boff, 0, t)),
        pl.BlockSpec((512, D1), lambda b, t: (0, 0)),
        pl.BlockSpec((512, D2), lambda b, t: (0, 0)),
        pl.BlockSpec((512, 1), lambda b, t: (0, 0)),
    ]
    args = [g6, g6, g6, g6, g6, g6, w4, points1, w0a, w0bp, b0c]
    aliases = {}
    if h0_prev is not None:
        in_specs.append(pl.BlockSpec(memory_space=pl.ANY))
        args.append(h0_prev)
        aliases = {len(args) - 1: 0}
    return pl.pallas_call(
        _k3_body,
        grid=(HB, NB3),
        in_specs=in_specs,
        out_specs=[
            pl.BlockSpec((512, NT3), lambda b, t: (0, (b + boff) * NB3 + t)),
            pl.BlockSpec((512, 8), lambda b, t: (0, 0)),
        ],
        out_shape=[
            jax.ShapeDtypeStruct((512, BN_PTS), jnp.bfloat16),
            jax.ShapeDtypeStruct((512, 8), jnp.float32),
        ],
        input_output_aliases=aliases,
    )(*args)


def _bn_ac(sa, sb, gam, bet):
    mean = (sa[:, 0:1] + sb[:, 0:1]) * INV_CNT
    ex2 = (sa[:, 1:2] + sb[:, 1:2]) * INV_CNT
    var = jnp.maximum(ex2 - mean * mean, 0.0)
    a = gam * lax.rsqrt(var + 1e-5)
    c = bet - mean * a
    return a, c


def _k4_body(h0_ref, sa_ref, sb_ref, gam_ref, bet_ref, w1_ref, b1_ref,
             h1_ref, sums_ref, ac_ref):
    @pl.when(pl.program_id(0) == 0)
    def _():
        a0, c0 = _bn_ac(sa_ref[...], sb_ref[...], gam_ref[...], bet_ref[...])
        ac_ref[:, 0:1] = a0
        ac_ref[:, 1:2] = c0

    a = ac_ref[:, 0:1]
    c = ac_ref[:, 1:2]
    h = h0_ref[...].astype(jnp.float32)
    x = jnp.maximum(a * h + c, 0.0)
    h1 = jnp.dot(w1_ref[...], x.astype(jnp.bfloat16),
                 preferred_element_type=jnp.float32)
    h1 = h1 + b1_ref[...]
    h1_ref[...] = h1.astype(jnp.bfloat16)

    @pl.when(pl.program_id(0) == 0)
    def _():
        sums_ref[...] = jnp.zeros_like(sums_ref)

    sums_ref[:, 0:1] += jnp.sum(h1, axis=1, keepdims=True)
    sums_ref[:, 1:2] += jnp.sum(h1 * h1, axis=1, keepdims=True)


def _mlp1(h0, s0a, s0b, gam0, bet0, w1, b1c):
    stat_spec = pl.BlockSpec((512, 8), lambda i: (0, 0))
    col_spec = pl.BlockSpec((512, 1), lambda i: (0, 0))
    return pl.pallas_call(
        _k4_body,
        grid=(BN_PTS // NT3,),
        in_specs=[
            pl.BlockSpec((512, NT3), lambda i: (0, i)),
            stat_spec, stat_spec, col_spec, col_spec,
            pl.BlockSpec((512, 512), lambda i: (0, 0)),
            col_spec,
        ],
        out_specs=[
            pl.BlockSpec((512, NT3), lambda i: (0, i)),
            pl.BlockSpec((512, 8), lambda i: (0, 0)),
        ],
        out_shape=[
            jax.ShapeDtypeStruct((512, BN_PTS), jnp.bfloat16),
            jax.ShapeDtypeStruct((512, 8), jnp.float32),
        ],
        scratch_shapes=[pltpu.VMEM((512, 2), jnp.float32)],
    )(h0, s0a, s0b, gam0, bet0, w1, b1c)


def _k5_body(h1_ref, s1_ref, gam_ref, bet_ref, out_ref, ac_ref):
    first = jnp.logical_and(pl.program_id(0) == 0, pl.program_id(1) == 0)

    @pl.when(first)
    def _():
        z = jnp.zeros_like(s1_ref[...])
        a1, c1 = _bn_ac(s1_ref[...], z, gam_ref[...], bet_ref[...])
        ac_ref[:, 0:1] = a1
        ac_ref[:, 1:2] = c1

    h = h1_ref[...].astype(jnp.float32)
    out_ref[0] = jnp.maximum(ac_ref[:, 0:1] * h + ac_ref[:, 1:2], 0.0)


def _final(h1, s1, gam1, bet1):
    col_spec = pl.BlockSpec((512, 1), lambda b, t: (0, 0))
    return pl.pallas_call(
        _k5_body,
        grid=(B, NB3),
        in_specs=[
            pl.BlockSpec((512, NT3), lambda b, t: (0, b * NB3 + t)),
            pl.BlockSpec((512, 8), lambda b, t: (0, 0)),
            col_spec, col_spec,
        ],
        out_specs=pl.BlockSpec((1, 512, NT3), lambda b, t: (b, 0, t)),
        out_shape=jax.ShapeDtypeStruct((B, 512, N), jnp.float32),
        scratch_shapes=[pltpu.VMEM((512, 2), jnp.float32)],
    )(h1, s1, gam1, bet1)


def kernel(xyz1, xyz2, points1, points2, W0, b0, g0, beta0,
           W1, b1, g1, beta1):
    x1t = jnp.transpose(xyz1, (0, 2, 1))
    x2p = jnp.concatenate(
        [xyz2, jnp.zeros((B, 5, S), xyz2.dtype)], axis=1)

    table = _make_table(points2)
    table2 = table.reshape(B * S * 2, HD)

    idx8a, w4a = _knn_weights(x1t, x2p, 0)
    ga = _gather_rows(table2, idx8a)
    idx8b, w4b = _knn_weights(x1t, x2p, HB)
    gb = _gather_rows(table2, idx8b)

    w0a = W0[:, :D1].astype(jnp.bfloat16)
    w0bp = W0[:, D1:].astype(jnp.bfloat16)
    b0c = b0.reshape(512, 1)

    h0a, s0a = _mlp0(ga.reshape(6, HPTS, HD), w4a, points1,
                     w0a, w0bp, b0c, 0)
    h0, s0b = _mlp0(gb.reshape(6, HPTS, HD), w4b, points1,
                    w0a, w0bp, b0c, HB, h0_prev=h0a)

    h1, s1 = _mlp1(h0, s0a, s0b, g0.reshape(512, 1), beta0.reshape(512, 1),
                   W1.astype(jnp.bfloat16), b1.reshape(512, 1))

    return _final(h1, s1, g1.reshape(512, 1), beta1.reshape(512, 1))


# speedup vs baseline: 1.5346x; 1.0110x over previous
import functools

import jax
import jax.numpy as jnp
from jax import lax
from jax.experimental import pallas as pl
from jax.experimental.pallas import tpu as pltpu
from jax.experimental.pallas import tpu_sc as plsc

B, N, S = 8, 4096, 1024
D1, D2 = 256, 512
HD = D2 // 2
BN_PTS = B * N
HB = B // 2
HPTS = HB * N
NT1 = 1024
NT3 = 2048
NT45 = 4096
GW = 128
NB1 = N // NT1
NB3 = N // NT3
INV_CNT = 1.0 / float(BN_PTS)


def _k0_body(p2_ref, t_ref):
    t_ref[...] = jnp.transpose(p2_ref[0])


def _make_table(points2):
    return pl.pallas_call(
        _k0_body,
        grid=(B, 2),
        in_specs=[pl.BlockSpec((1, D2, S // 2), lambda b, t: (b, 0, t))],
        out_specs=pl.BlockSpec((S // 2, D2), lambda b, t: (2 * b + t, 0)),
        out_shape=jax.ShapeDtypeStruct((B * S, D2), jnp.float32),
    )(points2)


def _k1_body(x1_ref, x2_ref, idx_ref, w_ref, *, boff):
    b = pl.program_id(0)
    x1 = x1_ref[0]
    x2 = x2_ref[0]
    x1b = x1.astype(jnp.bfloat16).astype(jnp.float32)
    x2b = x2.astype(jnp.bfloat16).astype(jnp.float32)
    acc = x1b[:, 0:1] * x2b[0:1, :]
    acc = acc + x1b[:, 1:2] * x2b[1:2, :]
    acc = acc + x1b[:, 2:3] * x2b[2:3, :]
    x1sq = jnp.sum(x1 * x1, axis=1, keepdims=True)
    x2sq = jnp.sum(x2 * x2, axis=0, keepdims=True)
    d = (-2.0 * acc + x1sq) + x2sq
    iota = lax.broadcasted_iota(jnp.int32, (NT1, S), 1)
    big = jnp.float32(3.0e38)
    vals, ids = [], []
    work = d
    for _ in range(3):
        m = jnp.min(work, axis=1, keepdims=True)
        sel = jnp.where(work == m, iota, jnp.int32(S))
        ik = jnp.min(sel, axis=1, keepdims=True)
        vals.append(m)
        ids.append(ik)
        work = jnp.where(iota == ik, big, work)
    r = [1.0 / (v + 1e-8) for v in vals]
    rs = r[0] + r[1] + r[2]
    w0 = r[0] / rs
    w1 = r[1] / rs
    w2 = r[2] / rs
    wz = jnp.zeros_like(w0)
    w_ref[0] = jnp.concatenate([w0, w1, w2, wz], axis=1)
    base = (b + boff) * S
    cols = []
    for k in range(3):
        gidx = 2 * (ids[k] + base)
        cols.append(gidx)
        cols.append(gidx + 1)
    zc = jnp.zeros_like(cols[0])
    idx_ref[...] = jnp.transpose(
        jnp.concatenate(cols + [zc, zc], axis=1))


def _knn_weights(x1t, x2p, boff):
    return pl.pallas_call(
        functools.partial(_k1_body, boff=boff),
        grid=(HB, NB1),
        in_specs=[
            pl.BlockSpec((1, NT1, 3), lambda b, t: (b + boff, t, 0)),
            pl.BlockSpec((1, 8, S), lambda b, t: (b + boff, 0, 0)),
        ],
        out_specs=[
            pl.BlockSpec((8, NT1), lambda b, t: (0, b * NB1 + t)),
            pl.BlockSpec((1, NT1, 4), lambda b, t: (b, t, 0)),
        ],
        out_shape=[
            jax.ShapeDtypeStruct((8, HPTS), jnp.int32),
            jax.ShapeDtypeStruct((HB, N, 4), jnp.float32),
        ],
    )(x1t, x2p)


def _gather_rows(table2, idx8):
    nwin = HPTS // GW
    mesh = plsc.VectorSubcoreMesh(core_axis_name="c", subcore_axis_name="s")

    @pl.kernel(
        out_type=jax.ShapeDtypeStruct((6 * HPTS, HD), jnp.float32),
        mesh=mesh,
    )
    def k(tab_hbm, i_hbm, o_hbm):
        def body(i_vmem, o_vmem):
            pltpu.sync_copy(tab_hbm.at[i_vmem.at[0]], o_vmem)

        pltpu.emit_pipeline(
            body,
            grid=(6, nwin),
            in_specs=[pl.BlockSpec((1, GW), lambda q, i: (q, i))],
            out_specs=[pl.BlockSpec((GW, HD), lambda q, i: (q * nwin + i, 0))],
            core_axis_name=("c", "s"),
            dimension_semantics=(pltpu.PARALLEL, pltpu.PARALLEL),
        )(i_hbm, o_hbm)

    return k(table2, idx8)


def _k3_body(g0l_ref, g0h_ref, g1l_ref, g1h_ref, g2l_ref, g2h_ref,
             w_ref, p1_ref, w0a_ref, w0bp_ref, b0_ref, *rest):
    h0_ref, sums_ref = rest[-2], rest[-1]
    p1b = p1_ref[0].astype(jnp.bfloat16)
    h = jnp.dot(w0a_ref[...], p1b, preferred_element_type=jnp.float32)
    w = w_ref[0]
    ilo = w[:, 0:1] * g0l_ref[0]
    ilo = ilo + w[:, 1:2] * g1l_ref[0]
    ilo = ilo + w[:, 2:3] * g2l_ref[0]
    ihi = w[:, 0:1] * g0h_ref[0]
    ihi = ihi + w[:, 1:2] * g1h_ref[0]
    ihi = ihi + w[:, 2:3] * g2h_ref[0]
    interp = jnp.concatenate([ilo, ihi], axis=1).astype(jnp.bfloat16)
    h = h + lax.dot_general(
        w0bp_ref[...], interp,
        (((1,), (1,)), ((), ())), preferred_element_type=jnp.float32)
    h = h + b0_ref[...]
    h0_ref[...] = h.astype(jnp.bfloat16)

    @pl.when(jnp.logical_and(pl.program_id(0) == 0, pl.program_id(1) == 0))
    def _():
        sums_ref[...] = jnp.zeros_like(sums_ref)

    sums_ref[:, 0:1] += jnp.sum(h, axis=1, keepdims=True)
    sums_ref[:, 1:2] += jnp.sum(h * h, axis=1, keepdims=True)


def _mlp0(g6, w4, points1, w0a, w0bp, b0c, boff, h0_prev=None):
    gspec = [pl.BlockSpec((1, NT3, HD),
                          (lambda q: (lambda b, t: (q, b * NB3 + t, 0)))(q))
             for q in range(6)]
    in_specs = gspec + [
        pl.BlockSpec((1, NT3, 4), lambda b, t: (b, t, 0)),
        pl.BlockSpec((1, D1, NT3), lambda b, t: (b + boff, 0, t)),
        pl.BlockSpec((512, D1), lambda b, t: (0, 0)),
        pl.BlockSpec((512, D2), lambda b, t: (0, 0)),
        pl.BlockSpec((512, 1), lambda b, t: (0, 0)),
    ]
    args = [g6, g6, g6, g6, g6, g6, w4, points1, w0a, w0bp, b0c]
    aliases = {}
    if h0_prev is not None:
        in_specs.append(pl.BlockSpec(memory_space=pl.ANY))
        args.append(h0_prev)
        aliases = {len(args) - 1: 0}
    return pl.pallas_call(
        _k3_body,
        grid=(HB, NB3),
        in_specs=in_specs,
        out_specs=[
            pl.BlockSpec((512, NT3), lambda b, t: (0, (b + boff) * NB3 + t)),
            pl.BlockSpec((512, 8), lambda b, t: (0, 0)),
        ],
        out_shape=[
            jax.ShapeDtypeStruct((512, BN_PTS), jnp.bfloat16),
            jax.ShapeDtypeStruct((512, 8), jnp.float32),
        ],
        input_output_aliases=aliases,
    )(*args)


def _bn_ac(sa, sb, gam, bet):
    mean = (sa[:, 0:1] + sb[:, 0:1]) * INV_CNT
    ex2 = (sa[:, 1:2] + sb[:, 1:2]) * INV_CNT
    var = jnp.maximum(ex2 - mean * mean, 0.0)
    a = gam * lax.rsqrt(var + 1e-5)
    c = bet - mean * a
    return a, c


def _k4_body(h0_ref, sa_ref, sb_ref, gam_ref, bet_ref, w1_ref, b1_ref,
             h1_ref, sums_ref, ac_ref):
    @pl.when(pl.program_id(0) == 0)
    def _():
        a0, c0 = _bn_ac(sa_ref[...], sb_ref[...], gam_ref[...], bet_ref[...])
        ac_ref[:, 0:1] = a0
        ac_ref[:, 1:2] = c0

    a = ac_ref[:, 0:1]
    c = ac_ref[:, 1:2]
    h = h0_ref[...].astype(jnp.float32)
    x = jnp.maximum(a * h + c, 0.0)
    h1 = jnp.dot(w1_ref[...], x.astype(jnp.bfloat16),
                 preferred_element_type=jnp.float32)
    h1 = h1 + b1_ref[...]
    h1_ref[...] = h1.astype(jnp.bfloat16)

    @pl.when(pl.program_id(0) == 0)
    def _():
        sums_ref[...] = jnp.zeros_like(sums_ref)

    sums_ref[:, 0:1] += jnp.sum(h1, axis=1, keepdims=True)
    sums_ref[:, 1:2] += jnp.sum(h1 * h1, axis=1, keepdims=True)


def _mlp1(h0, s0a, s0b, gam0, bet0, w1, b1c):
    stat_spec = pl.BlockSpec((512, 8), lambda i: (0, 0))
    col_spec = pl.BlockSpec((512, 1), lambda i: (0, 0))
    return pl.pallas_call(
        _k4_body,
        grid=(BN_PTS // NT45,),
        in_specs=[
            pl.BlockSpec((512, NT45), lambda i: (0, i)),
            stat_spec, stat_spec, col_spec, col_spec,
            pl.BlockSpec((512, 512), lambda i: (0, 0)),
            col_spec,
        ],
        out_specs=[
            pl.BlockSpec((512, NT45), lambda i: (0, i)),
            pl.BlockSpec((512, 8), lambda i: (0, 0)),
        ],
        out_shape=[
            jax.ShapeDtypeStruct((512, BN_PTS), jnp.bfloat16),
            jax.ShapeDtypeStruct((512, 8), jnp.float32),
        ],
        scratch_shapes=[pltpu.VMEM((512, 2), jnp.float32)],
    )(h0, s0a, s0b, gam0, bet0, w1, b1c)


def _k5_body(h1_ref, s1_ref, gam_ref, bet_ref, out_ref, ac_ref):
    first = jnp.logical_and(pl.program_id(0) == 0, pl.program_id(1) == 0)

    @pl.when(first)
    def _():
        z = jnp.zeros_like(s1_ref[...])
        a1, c1 = _bn_ac(s1_ref[...], z, gam_ref[...], bet_ref[...])
        ac_ref[:, 0:1] = a1
        ac_ref[:, 1:2] = c1

    h = h1_ref[...].astype(jnp.float32)
    out_ref[0] = jnp.maximum(ac_ref[:, 0:1] * h + ac_ref[:, 1:2], 0.0)


def _final(h1, s1, gam1, bet1):
    col_spec = pl.BlockSpec((512, 1), lambda b, t: (0, 0))
    return pl.pallas_call(
        _k5_body,
        grid=(B, N // NT45),
        in_specs=[
            pl.BlockSpec((512, NT45),
                         lambda b, t: (0, b * (N // NT45) + t)),
            pl.BlockSpec((512, 8), lambda b, t: (0, 0)),
            col_spec, col_spec,
        ],
        out_specs=pl.BlockSpec((1, 512, NT45), lambda b, t: (b, 0, t)),
        out_shape=jax.ShapeDtypeStruct((B, 512, N), jnp.float32),
        scratch_shapes=[pltpu.VMEM((512, 2), jnp.float32)],
    )(h1, s1, gam1, bet1)


def kernel(xyz1, xyz2, points1, points2, W0, b0, g0, beta0,
           W1, b1, g1, beta1):
    x1t = jnp.transpose(xyz1, (0, 2, 1))
    x2p = jnp.concatenate(
        [xyz2, jnp.zeros((B, 5, S), xyz2.dtype)], axis=1)

    table = _make_table(points2)
    table2 = table.reshape(B * S * 2, HD)

    idx8a, w4a = _knn_weights(x1t, x2p, 0)
    ga = _gather_rows(table2, idx8a)
    idx8b, w4b = _knn_weights(x1t, x2p, HB)
    gb = _gather_rows(table2, idx8b)

    w0a = W0[:, :D1].astype(jnp.bfloat16)
    w0bp = W0[:, D1:].astype(jnp.bfloat16)
    b0c = b0.reshape(512, 1)

    h0a, s0a = _mlp0(ga.reshape(6, HPTS, HD), w4a, points1,
                     w0a, w0bp, b0c, 0)
    h0, s0b = _mlp0(gb.reshape(6, HPTS, HD), w4b, points1,
                    w0a, w0bp, b0c, HB, h0_prev=h0a)

    h1, s1 = _mlp1(h0, s0a, s0b, g0.reshape(512, 1), beta0.reshape(512, 1),
                   W1.astype(jnp.bfloat16), b1.reshape(512, 1))

    return _final(h1, s1, g1.reshape(512, 1), beta1.reshape(512, 1))
